# baseline (speedup 1.0000x reference)
import functools

import jax
import jax.numpy as jnp
from jax import lax
from jax.experimental import pallas as pl
from jax.experimental.pallas import tpu as pltpu
from jax.experimental.pallas import tpu_sc as plsc

I_, K_, B_, D_ = 8, 4096, 1024, 128
KB = 1024
NKB = K_ // KB
_PREC = lax.Precision.DEFAULT


CH = 512
NCH = K_ // CH


def _argmin_body(x_ref, F_ref, out_ref):
    i = pl.program_id(0)
    x = x_ref[...]
    xs = x * -2.0
    x2 = jnp.sum(x * x, axis=1, keepdims=True).reshape(1, B_)

    R = 8
    acc_v, acc_i = None, None
    for c in range(NCH):
        Fc = F_ref[c * CH:(c + 1) * CH, :]
        f2c = jnp.sum(Fc * Fc, axis=1, keepdims=True)
        xfc = lax.dot_general(Fc, xs, (((1,), (1,)), ((), ())),
                              precision=_PREC,
                              preferred_element_type=jnp.float32)
        nr = CH // R
        for r in range(0, nr, 2):
            r0 = c * nr + r
            d0 = (x2 + f2c[r * R:(r + 1) * R, :]) + xfc[r * R:(r + 1) * R, :]
            d1 = (x2 + f2c[(r + 1) * R:(r + 2) * R, :]) + xfc[(r + 1) * R:(r + 2) * R, :]
            v01 = jnp.minimum(d0, d1)
            i01 = jnp.where(d1 < d0, jnp.float32(r0 + 1), jnp.float32(r0))
            if acc_v is None:
                acc_v, acc_i = v01, i01
            else:
                m = v01 < acc_v
                acc_v = jnp.minimum(acc_v, v01)
                acc_i = jnp.where(m, i01, acc_i)

    kf = acc_i * jnp.float32(R) + lax.broadcasted_iota(
        jnp.int32, (R, B_), 0).astype(jnp.float32)

    def comb(vl, kl, vh, kh):
        better = (vh < vl) | ((vh == vl) & (kh < kl))
        return jnp.where(better, vh, vl), jnp.where(better, kh, kl)

    v, k = comb(acc_v[0:4, :], kf[0:4, :], acc_v[4:8, :], kf[4:8, :])
    v, k = comb(v[0:2, :], k[0:2, :], v[2:4, :], k[2:4, :])
    v, k = comb(v[0:1, :], k[0:1, :], v[1:2, :], k[1:2, :])
    out_ref[...] = k[0, :].astype(jnp.int32) + i * K_


_argmin_call = pl.pallas_call(
    _argmin_body,
    grid=(I_,),
    in_specs=[
        pl.BlockSpec((None, B_, D_), lambda i: (i, 0, 0)),
        pl.BlockSpec((None, K_, D_), lambda i: (i, 0, 0)),
    ],
    out_specs=pl.BlockSpec((B_,), lambda i: (i,)),
    out_shape=jax.ShapeDtypeStruct((I_ * B_,), jnp.int32),
)

_NC, _NS = 2, 16
_NW = _NC * _NS
_BT = I_ * B_
_BPW = _BT // _NW


@functools.cache
def _sc_gather_fn():
    mesh = plsc.VectorSubcoreMesh(core_axis_name="c", subcore_axis_name="s")

    @functools.partial(
        pl.kernel,
        mesh=mesh,
        out_type=jax.ShapeDtypeStruct((_BT, D_), jnp.float32),
        scratch_types=[
            pltpu.VMEM((_BPW,), jnp.int32),
            pltpu.VMEM((_BPW, D_), jnp.float32),
            pltpu.SemaphoreType.DMA,
        ],
    )
    def _sc_gather(table_hbm, idx_hbm, out_hbm, idx_v, rows_v, sem):
        wid = lax.axis_index("s") * _NC + lax.axis_index("c")
        base = wid * _BPW
        pltpu.sync_copy(idx_hbm.at[pl.ds(base, _BPW)], idx_v)
        pltpu.async_copy(table_hbm.at[idx_v], rows_v, sem).wait()
        pltpu.sync_copy(rows_v, out_hbm.at[pl.ds(base, _BPW)])

    return _sc_gather


def kernel(F, x):
    idxf = _argmin_call(x, F)
    out = _sc_gather_fn()(F.reshape(I_ * K_, D_), idxf)
    return out.reshape(I_, B_, D_)


# speedup vs baseline: 84.6137x; 84.6137x over previous
import functools

import jax
import jax.numpy as jnp
from jax import lax
from jax.experimental import pallas as pl
from jax.experimental.pallas import tpu as pltpu
from jax.experimental.pallas import tpu_sc as plsc

I_, K_, B_, D_ = 8, 4096, 1024, 128
KB = 1024
NKB = K_ // KB
_PREC = lax.Precision.DEFAULT


CH = 512
NCH = K_ // CH


def _argmin_body(x_ref, F_ref, out_ref):
    i = pl.program_id(0)
    x = x_ref[...]
    xs = x * -2.0
    x2 = jnp.sum(x * x, axis=1, keepdims=True)

    chunks = []
    for c in range(NCH):
        Fc = F_ref[c * CH:(c + 1) * CH, :]
        f2c = jnp.sum(Fc * Fc, axis=1)[None, :]
        xfc = lax.dot_general(xs, Fc, (((1,), (1,)), ((), ())),
                              precision=_PREC,
                              preferred_element_type=jnp.float32)
        chunks.append((f2c, xfc))

    acc_v, acc_i = None, None
    for c, (f2c, xfc) in enumerate(chunks):
        ds = [(x2 + f2c[:, s * 128:(s + 1) * 128]) + xfc[:, s * 128:(s + 1) * 128]
              for s in range(CH // 128)]
        s0 = c * (CH // 128)
        v01 = jnp.minimum(ds[0], ds[1])
        i01 = jnp.where(ds[1] < ds[0], jnp.float32(s0 + 1), jnp.float32(s0))
        v23 = jnp.minimum(ds[2], ds[3])
        i23 = jnp.where(ds[3] < ds[2], jnp.float32(s0 + 3), jnp.float32(s0 + 2))
        vq = jnp.minimum(v01, v23)
        iq = jnp.where(v23 < v01, i23, i01)
        if acc_v is None:
            acc_v, acc_i = vq, iq
        else:
            ma = vq < acc_v
            acc_v = jnp.minimum(acc_v, vq)
            acc_i = jnp.where(ma, iq, acc_i)

    big = jnp.float32(2.0 ** 30)
    iotaf = lax.broadcasted_iota(jnp.int32, (B_, 128), 1).astype(jnp.float32)
    gmin = jnp.min(acc_v, axis=1, keepdims=True)
    cand = jnp.where(acc_v == gmin, acc_i * 128.0 + iotaf, big)
    midx = jnp.min(cand, axis=1, keepdims=True).astype(jnp.int32)
    out_ref[...] = midx[:, 0] + i * K_


_argmin_call = pl.pallas_call(
    _argmin_body,
    grid=(I_,),
    in_specs=[
        pl.BlockSpec((None, B_, D_), lambda i: (i, 0, 0)),
        pl.BlockSpec((None, K_, D_), lambda i: (i, 0, 0)),
    ],
    out_specs=pl.BlockSpec((B_,), lambda i: (i,)),
    out_shape=jax.ShapeDtypeStruct((I_ * B_,), jnp.int32),
)

_NC, _NS = 2, 16
_NW = _NC * _NS
_BT = I_ * B_
_BPW = _BT // _NW


@functools.cache
def _sc_gather_fn():
    mesh = plsc.VectorSubcoreMesh(core_axis_name="c", subcore_axis_name="s")

    @functools.partial(
        pl.kernel,
        mesh=mesh,
        out_type=jax.ShapeDtypeStruct((_BT, D_), jnp.float32),
        scratch_types=[
            pltpu.VMEM((_BPW,), jnp.int32),
            pltpu.VMEM((_BPW, D_), jnp.float32),
            pltpu.SemaphoreType.DMA,
        ],
    )
    def _sc_gather(table_hbm, idx_hbm, out_hbm, idx_v, rows_v, sem):
        wid = lax.axis_index("s") * _NC + lax.axis_index("c")
        base = wid * _BPW
        pltpu.sync_copy(idx_hbm.at[pl.ds(base, _BPW)], idx_v)
        pltpu.async_copy(table_hbm.at[idx_v], rows_v, sem).wait()
        pltpu.sync_copy(rows_v, out_hbm.at[pl.ds(base, _BPW)])

    return _sc_gather


def kernel(F, x):
    idxf = _argmin_call(x, F)
    out = _sc_gather_fn()(F.reshape(I_ * K_, D_), idxf)
    return out.reshape(I_, B_, D_)


# speedup vs baseline: 85.6579x; 1.0123x over previous
import functools

import jax
import jax.numpy as jnp
from jax import lax
from jax.experimental import pallas as pl
from jax.experimental.pallas import tpu as pltpu
from jax.experimental.pallas import tpu_sc as plsc

I_, K_, B_, D_ = 8, 4096, 1024, 128
KB = 1024
NKB = K_ // KB
_PREC = lax.Precision.DEFAULT


CH = 1024
NCH = K_ // CH


def _argmin_body(x_ref, F_ref, out_ref):
    i = pl.program_id(0)
    x = x_ref[...]
    xs = x * -2.0
    x2 = jnp.sum(x * x, axis=1, keepdims=True)

    chunks = []
    for c in range(NCH):
        Fc = F_ref[c * CH:(c + 1) * CH, :]
        f2c = jnp.sum(Fc * Fc, axis=1)[None, :]
        xfc = lax.dot_general(xs, Fc, (((1,), (1,)), ((), ())),
                              precision=_PREC,
                              preferred_element_type=jnp.float32)
        chunks.append((f2c, xfc))

    acc_v, acc_i = None, None
    for c, (f2c, xfc) in enumerate(chunks):
        ds = [(x2 + f2c[:, s * 128:(s + 1) * 128]) + xfc[:, s * 128:(s + 1) * 128]
              for s in range(CH // 128)]
        s0 = c * (CH // 128)
        vs = [(d, jnp.float32(s0 + s)) for s, d in enumerate(ds)]
        while len(vs) > 1:
            nxt = []
            for (v0, i0), (v1, i1) in zip(vs[0::2], vs[1::2]):
                m = v1 < v0
                nxt.append((jnp.minimum(v0, v1), jnp.where(m, i1, i0)))
            vs = nxt
        vq, iq = vs[0]
        if acc_v is None:
            acc_v, acc_i = vq, iq
        else:
            ma = vq < acc_v
            acc_v = jnp.minimum(acc_v, vq)
            acc_i = jnp.where(ma, iq, acc_i)

    big = jnp.float32(2.0 ** 30)
    iotaf = lax.broadcasted_iota(jnp.int32, (B_, 128), 1).astype(jnp.float32)
    gmin = jnp.min(acc_v, axis=1, keepdims=True)
    cand = jnp.where(acc_v == gmin, acc_i * 128.0 + iotaf, big)
    midx = jnp.min(cand, axis=1, keepdims=True).astype(jnp.int32)
    out_ref[...] = midx[:, 0] + i * K_


_argmin_call = pl.pallas_call(
    _argmin_body,
    grid=(I_,),
    in_specs=[
        pl.BlockSpec((None, B_, D_), lambda i: (i, 0, 0)),
        pl.BlockSpec((None, K_, D_), lambda i: (i, 0, 0)),
    ],
    out_specs=pl.BlockSpec((B_,), lambda i: (i,)),
    out_shape=jax.ShapeDtypeStruct((I_ * B_,), jnp.int32),
)

_NC, _NS = 2, 16
_NW = _NC * _NS
_BT = I_ * B_
_BPW = _BT // _NW


@functools.cache
def _sc_gather_fn():
    mesh = plsc.VectorSubcoreMesh(core_axis_name="c", subcore_axis_name="s")

    @functools.partial(
        pl.kernel,
        mesh=mesh,
        out_type=jax.ShapeDtypeStruct((_BT, D_), jnp.float32),
        scratch_types=[
            pltpu.VMEM((_BPW,), jnp.int32),
            pltpu.VMEM((_BPW, D_), jnp.float32),
            pltpu.SemaphoreType.DMA,
        ],
    )
    def _sc_gather(table_hbm, idx_hbm, out_hbm, idx_v, rows_v, sem):
        wid = lax.axis_index("s") * _NC + lax.axis_index("c")
        base = wid * _BPW
        pltpu.sync_copy(idx_hbm.at[pl.ds(base, _BPW)], idx_v)
        pltpu.async_copy(table_hbm.at[idx_v], rows_v, sem).wait()
        pltpu.sync_copy(rows_v, out_hbm.at[pl.ds(base, _BPW)])

    return _sc_gather


def kernel(F, x):
    idxf = _argmin_call(x, F)
    out = _sc_gather_fn()(F.reshape(I_ * K_, D_), idxf)
    return out.reshape(I_, B_, D_)


# speedup vs baseline: 86.5216x; 1.0101x over previous
import functools

import jax
import jax.numpy as jnp
from jax import lax
from jax.experimental import pallas as pl
from jax.experimental.pallas import tpu as pltpu
from jax.experimental.pallas import tpu_sc as plsc

I_, K_, B_, D_ = 8, 4096, 1024, 128
KB = 1024
NKB = K_ // KB
_PREC = lax.Precision.DEFAULT


CH = 4096
NCH = K_ // CH


def _argmin_body(x_ref, F_ref, out_ref):
    i = pl.program_id(0)
    x = x_ref[...]
    xs = x * -2.0
    x2 = jnp.sum(x * x, axis=1, keepdims=True)

    chunks = []
    for c in range(NCH):
        Fc = F_ref[c * CH:(c + 1) * CH, :]
        f2c = jnp.sum(Fc * Fc, axis=1)[None, :]
        xfc = lax.dot_general(xs, Fc, (((1,), (1,)), ((), ())),
                              precision=_PREC,
                              preferred_element_type=jnp.float32)
        chunks.append((f2c, xfc))

    acc_v, acc_i = None, None
    for c, (f2c, xfc) in enumerate(chunks):
        ds = [(x2 + f2c[:, s * 128:(s + 1) * 128]) + xfc[:, s * 128:(s + 1) * 128]
              for s in range(CH // 128)]
        s0 = c * (CH // 128)
        vs = [(d, jnp.float32(s0 + s)) for s, d in enumerate(ds)]
        while len(vs) > 1:
            nxt = []
            for (v0, i0), (v1, i1) in zip(vs[0::2], vs[1::2]):
                m = v1 < v0
                nxt.append((jnp.minimum(v0, v1), jnp.where(m, i1, i0)))
            vs = nxt
        vq, iq = vs[0]
        if acc_v is None:
            acc_v, acc_i = vq, iq
        else:
            ma = vq < acc_v
            acc_v = jnp.minimum(acc_v, vq)
            acc_i = jnp.where(ma, iq, acc_i)

    big = jnp.float32(2.0 ** 30)
    iotaf = lax.broadcasted_iota(jnp.int32, (B_, 128), 1).astype(jnp.float32)
    gmin = jnp.min(acc_v, axis=1, keepdims=True)
    cand = jnp.where(acc_v == gmin, acc_i * 128.0 + iotaf, big)
    midx = jnp.min(cand, axis=1, keepdims=True).astype(jnp.int32)
    out_ref[...] = midx[:, 0] + i * K_


_argmin_call = pl.pallas_call(
    _argmin_body,
    grid=(I_,),
    in_specs=[
        pl.BlockSpec((None, B_, D_), lambda i: (i, 0, 0)),
        pl.BlockSpec((None, K_, D_), lambda i: (i, 0, 0)),
    ],
    out_specs=pl.BlockSpec((B_,), lambda i: (i,)),
    out_shape=jax.ShapeDtypeStruct((I_ * B_,), jnp.int32),
)

_NC, _NS = 2, 16
_NW = _NC * _NS
_BT = I_ * B_
_BPW = _BT // _NW


@functools.cache
def _sc_gather_fn():
    mesh = plsc.VectorSubcoreMesh(core_axis_name="c", subcore_axis_name="s")

    @functools.partial(
        pl.kernel,
        mesh=mesh,
        out_type=jax.ShapeDtypeStruct((_BT, D_), jnp.float32),
        scratch_types=[
            pltpu.VMEM((_BPW,), jnp.int32),
            pltpu.VMEM((_BPW, D_), jnp.float32),
            pltpu.SemaphoreType.DMA,
        ],
    )
    def _sc_gather(table_hbm, idx_hbm, out_hbm, idx_v, rows_v, sem):
        wid = lax.axis_index("s") * _NC + lax.axis_index("c")
        base = wid * _BPW
        pltpu.sync_copy(idx_hbm.at[pl.ds(base, _BPW)], idx_v)
        pltpu.async_copy(table_hbm.at[idx_v], rows_v, sem).wait()
        pltpu.sync_copy(rows_v, out_hbm.at[pl.ds(base, _BPW)])

    return _sc_gather


def kernel(F, x):
    idxf = _argmin_call(x, F)
    out = _sc_gather_fn()(F.reshape(I_ * K_, D_), idxf)
    return out.reshape(I_, B_, D_)
